# baseline (speedup 1.0000x reference)
import functools

import jax
import jax.numpy as jnp
from jax import lax
from jax.experimental import pallas as pl
from jax.experimental.pallas import tpu as pltpu
from jax.experimental.pallas import tpu_sc as plsc

_SCALE = 0.1
_K = 8
_L = 5
_P = 1024
_D = 768
_B = 128
_S = 196
_OUT_S = _K * _L + _S
_HEAD = _K * _L
_ROW_W = _L * _D

_NC = 2
_NS = 16
_NW = _NC * _NS
_ROWS_PER_W = _B // _NW
_CHUNKS = _P // 16


_ST = 4


def _map_to_ball(u):
    ss = jnp.sum(u * u, axis=-1, keepdims=True)
    un = u * lax.rsqrt(jnp.maximum(ss, 1e-12))
    us = un * _SCALE
    n2 = jnp.sum(us * us, axis=-1, keepdims=True)
    n = jnp.sqrt(jnp.maximum(n2, 1e-15))
    y = jnp.tanh(n) * us / n
    yn2 = jnp.sum(y * y, axis=-1, keepdims=True)
    ynorm = jnp.sqrt(jnp.maximum(yn2, 1e-15))
    maxnorm = 1.0 - 4e-3
    return jnp.where(ynorm > maxnorm, y / ynorm * maxnorm, y)


def _mean_copy_dist_body(x_ref, pk_ref, big_ref, sim_ref, yn_ref, sum_ref):
    j = pl.program_id(0)
    xb = x_ref[...]
    big_ref[...] = xb

    @pl.when(j == 0)
    def _():
        sum_ref[...] = jnp.zeros((_B, _D), jnp.float32)

    sum_ref[...] += jnp.sum(xb, axis=0)

    @pl.when(j == _S // _ST - 1)
    def _():
        x = _map_to_ball(sum_ref[...] / float(_S))
        y = _map_to_ball(pk_ref[...])
        yn_ref[...] = y
        x2 = jnp.sum(x * x, axis=-1, keepdims=True)
        y2 = jnp.sum(y * y, axis=-1)[None, :]
        xy = lax.dot_general(x, y, (((1,), (1,)), ((), ())),
                             precision=lax.Precision.HIGHEST,
                             preferred_element_type=jnp.float32)
        alpha = 1.0 - 2.0 * xy + y2
        beta = 1.0 - x2
        num2 = alpha * alpha * x2 + beta * beta * y2 - 2.0 * alpha * beta * xy
        den = 1.0 - 2.0 * xy + x2 * y2
        norm = jnp.sqrt(jnp.maximum(num2, 1e-15)) / (den + 1e-15)
        arg = jnp.clip(norm, 0.0, 1.0 - 1e-7)
        dist = jnp.log((1.0 + arg) / (1.0 - arg))
        sim_ref[...] = -dist


def _run_mean_copy_dist(xT, prompt_key):
    return pl.pallas_call(
        _mean_copy_dist_body,
        grid=(_S // _ST,),
        in_specs=[
            pl.BlockSpec((_ST, _B, _D), lambda j: (j, 0, 0)),
            pl.BlockSpec((_P, _D), lambda j: (0, 0)),
        ],
        out_specs=[
            pl.BlockSpec((_ST, _B, _D), lambda j: (j + _HEAD // _ST, 0, 0)),
            pl.BlockSpec((_B, _P), lambda j: (0, 0)),
            pl.BlockSpec((_P, _D), lambda j: (0, 0)),
        ],
        out_shape=[
            jax.ShapeDtypeStruct((_OUT_S, _B, _D), jnp.float32),
            jax.ShapeDtypeStruct((_B, _P), jnp.float32),
            jax.ShapeDtypeStruct((_P, _D), jnp.float32),
        ],
        scratch_shapes=[pltpu.VMEM((_B, _D), jnp.float32)],
    )(xT, prompt_key)


def _vgather16(v, i):
    dn = lax.GatherDimensionNumbers(offset_dims=(), collapsed_slice_dims=(0,),
                                    start_index_map=(0,))
    return lax.gather(v, i[:, None], dimension_numbers=dn, slice_sizes=(1,),
                      mode=lax.GatherScatterMode.PROMISE_IN_BOUNDS)


def _sc_body(sim_hbm, prompt_hbm, key_hbm,
             head_hbm, idx_hbm, bkn_hbm, part_hbm,
             sim_v, idx128_v, idx40_v, rows_v, bkn_v, sum_v, sem, sem2):
    cid = lax.axis_index("c")
    sid = lax.axis_index("s")
    wid = cid * _NS + sid
    lane = lax.iota(jnp.int32, 16)
    neg = jnp.full((16,), -3.0e38, jnp.float32)
    zidx = jnp.zeros((16,), jnp.int32)
    zf = jnp.zeros((16,), jnp.float32)

    sevens = jnp.full((16,), 7, jnp.int32)
    pltpu.sync_copy(sim_hbm.at[pl.ds(wid * _ROWS_PER_W, _ROWS_PER_W)], sim_v)

    def row_body(r, acc):
        b = wid * _ROWS_PER_W + r

        def chunk_body(i, carry):
            bk, bi, t = carry
            ck = sim_v[r, pl.ds(i * 16, 16)]
            cmax = jnp.max(ck)

            def merge(args):
                bk0, bi0 = args
                ci = lane + i * 16
                sck, sci = plsc.sort_key_val(ck, ci, descending=True)
                rk = lax.rev(sck, (0,))
                ri = lax.rev(sci, (0,))
                mk = jnp.where(lane < 8, bk0, rk)
                mi = jnp.where(lane < 8, bi0, ri)
                bk2, bi2 = plsc.sort_key_val(mk, mi, descending=True)
                t2 = jnp.max(_vgather16(bk2, sevens))
                return (bk2, bi2, t2)

            return lax.cond(cmax > t, merge, lambda a: (a[0], a[1], t),
                            (bk, bi))

        bk, bi, _ = lax.fori_loop(
            0, _CHUNKS, chunk_body,
            (neg, zidx, jnp.float32(-3.0e38)))
        iv = jnp.where(lane < 8, bi, jnp.int32(2147483647))
        fi, fv = plsc.sort_key_val(iv, bk, descending=False)
        idx128_v[pl.ds(0, 16)] = fi
        pltpu.sync_copy(idx128_v, idx_hbm.at[b])
        for c in range(3):
            j = lane + 16 * c
            q = lax.div(j, jnp.int32(_L))
            s = j - q * _L
            sel = _vgather16(fi, jnp.minimum(q, jnp.int32(15)))
            idx40_v[pl.ds(16 * c, 16)] = s * _P + sel
        pltpu.async_copy(prompt_hbm.at[idx40_v.at[pl.ds(0, _HEAD)]],
                         rows_v, sem).wait()
        pltpu.sync_copy(rows_v, head_hbm.at[b])
        pltpu.async_copy(key_hbm.at[idx128_v.at[pl.ds(0, 8)]],
                         bkn_v, sem2).wait()
        pltpu.sync_copy(bkn_v, bkn_hbm.at[b])
        sel_sim = jnp.where(lane < 8, fv, jnp.float32(0.0))
        return acc - jnp.sum(sel_sim)

    acc = lax.fori_loop(0, _ROWS_PER_W, row_body, jnp.float32(0.0))
    for c in range(8):
        sum_v[pl.ds(16 * c, 16)] = zf
    sum_v[pl.ds(0, 16)] = jnp.where(lane == 0, jnp.full((16,), acc), zf)
    pltpu.sync_copy(sum_v, part_hbm.at[wid])


def _run_topk_gather(sim, prompt_flat, yn):
    mesh = plsc.VectorSubcoreMesh(core_axis_name="c", subcore_axis_name="s",
                                  num_cores=_NC, num_subcores=_NS)
    fn = pl.kernel(
        _sc_body,
        out_type=[
            jax.ShapeDtypeStruct((_B, _HEAD, _D), jnp.float32),
            jax.ShapeDtypeStruct((_B, 128), jnp.int32),
            jax.ShapeDtypeStruct((_B, _K, _D), jnp.float32),
            jax.ShapeDtypeStruct((_NW, 128), jnp.float32),
        ],
        mesh=mesh,
        compiler_params=pltpu.CompilerParams(needs_layout_passes=False,
                                             use_tc_tiling_on_sc=True),
        scratch_types=[
            pltpu.VMEM((_ROWS_PER_W, _P), jnp.float32),
            pltpu.VMEM((128,), jnp.int32),
            pltpu.VMEM((48,), jnp.int32),
            pltpu.VMEM((_HEAD, _D), jnp.float32),
            pltpu.VMEM((_K, _D), jnp.float32),
            pltpu.VMEM((128,), jnp.float32),
            pltpu.SemaphoreType.DMA,
            pltpu.SemaphoreType.DMA,
        ],
    )
    return fn(sim, prompt_flat, yn)


_DBT = 16


def _assemble_body(big_in_ref, head_ref, part_ref, big_ref, rs_ref):
    del big_in_ref
    big_ref[...] = jnp.transpose(head_ref[...], (1, 0, 2))

    @pl.when(pl.program_id(0) == 0)
    def _():
        rs_ref[...] = jnp.sum(part_ref[...]).reshape(1, 1) / float(_B)


def _run_assemble(bigT0, head, part):
    return pl.pallas_call(
        _assemble_body,
        grid=(_B // _DBT,),
        in_specs=[
            pl.BlockSpec(memory_space=pl.ANY),
            pl.BlockSpec((_DBT, _HEAD, _D), lambda b: (b, 0, 0)),
            pl.BlockSpec((_NW, 128), lambda b: (0, 0)),
        ],
        out_specs=[
            pl.BlockSpec((_HEAD, _DBT, _D), lambda b: (0, b, 0)),
            pl.BlockSpec((1, 1), lambda b: (0, 0)),
        ],
        out_shape=[
            jax.ShapeDtypeStruct((_OUT_S, _B, _D), jnp.float32),
            jax.ShapeDtypeStruct((1, 1), jnp.float32),
        ],
        input_output_aliases={0: 0},
    )(bigT0, head, part)


def kernel(x_embed, prompt, prompt_key):
    xT = jnp.transpose(x_embed, (1, 0, 2))
    bigT0, sim, yn = _run_mean_copy_dist(xT, prompt_key)
    prompt_flat = jnp.transpose(prompt, (1, 0, 2)).reshape(_L * _P, _D)
    head, idx_pad, bkn, part = _run_topk_gather(sim, prompt_flat, yn)
    bigT, rs = _run_assemble(bigT0, head, part)
    big = jnp.transpose(bigT, (1, 0, 2))
    return big, rs[0, 0], sim, idx_pad[:, :_K], bkn


# speedup vs baseline: 1.0535x; 1.0535x over previous
import functools

import jax
import jax.numpy as jnp
from jax import lax
from jax.experimental import pallas as pl
from jax.experimental.pallas import tpu as pltpu
from jax.experimental.pallas import tpu_sc as plsc

_SCALE = 0.1
_K = 8
_L = 5
_P = 1024
_D = 768
_B = 128
_S = 196
_OUT_S = _K * _L + _S
_HEAD = _K * _L
_ROW_W = _L * _D

_NC = 2
_NS = 16
_NW = _NC * _NS
_ROWS_PER_W = _B // _NW
_CHUNKS = _P // 16


_ST = 4


def _map_to_ball(u):
    ss = jnp.sum(u * u, axis=-1, keepdims=True)
    un = u * lax.rsqrt(jnp.maximum(ss, 1e-12))
    us = un * _SCALE
    n2 = jnp.sum(us * us, axis=-1, keepdims=True)
    n = jnp.sqrt(jnp.maximum(n2, 1e-15))
    y = jnp.tanh(n) * us / n
    yn2 = jnp.sum(y * y, axis=-1, keepdims=True)
    ynorm = jnp.sqrt(jnp.maximum(yn2, 1e-15))
    maxnorm = 1.0 - 4e-3
    return jnp.where(ynorm > maxnorm, y / ynorm * maxnorm, y)


def _mean_copy_dist_body(x_ref, pk_ref, big_ref, sim_ref, yn_ref, sum_ref):
    j = pl.program_id(0)
    xb = x_ref[...]
    big_ref[...] = xb

    @pl.when(j == 0)
    def _():
        sum_ref[...] = jnp.zeros((_B, _D), jnp.float32)

    sum_ref[...] += jnp.sum(xb, axis=0)

    @pl.when(j == _S // _ST - 1)
    def _():
        x = _map_to_ball(sum_ref[...] / float(_S))
        y = _map_to_ball(pk_ref[...])
        yn_ref[...] = y
        x2 = jnp.sum(x * x, axis=-1, keepdims=True)
        y2 = jnp.sum(y * y, axis=-1)[None, :]
        xy = lax.dot_general(x, y, (((1,), (1,)), ((), ())),
                             precision=lax.Precision.HIGHEST,
                             preferred_element_type=jnp.float32)
        alpha = 1.0 - 2.0 * xy + y2
        beta = 1.0 - x2
        num2 = alpha * alpha * x2 + beta * beta * y2 - 2.0 * alpha * beta * xy
        den = 1.0 - 2.0 * xy + x2 * y2
        norm = jnp.sqrt(jnp.maximum(num2, 1e-15)) / (den + 1e-15)
        arg = jnp.clip(norm, 0.0, 1.0 - 1e-7)
        dist = jnp.log((1.0 + arg) / (1.0 - arg))
        sim_ref[...] = -dist


def _run_mean_copy_dist(xT, prompt_key):
    return pl.pallas_call(
        _mean_copy_dist_body,
        grid=(_S // _ST,),
        in_specs=[
            pl.BlockSpec((_ST, _B, _D), lambda j: (j, 0, 0)),
            pl.BlockSpec((_P, _D), lambda j: (0, 0)),
        ],
        out_specs=[
            pl.BlockSpec((_ST, _B, _D), lambda j: (j + _HEAD // _ST, 0, 0)),
            pl.BlockSpec((_B, _P), lambda j: (0, 0)),
            pl.BlockSpec((_P, _D), lambda j: (0, 0)),
        ],
        out_shape=[
            jax.ShapeDtypeStruct((_OUT_S, _B, _D), jnp.float32),
            jax.ShapeDtypeStruct((_B, _P), jnp.float32),
            jax.ShapeDtypeStruct((_P, _D), jnp.float32),
        ],
        scratch_shapes=[pltpu.VMEM((_B, _D), jnp.float32)],
    )(xT, prompt_key)


def _vgather16(v, i):
    dn = lax.GatherDimensionNumbers(offset_dims=(), collapsed_slice_dims=(0,),
                                    start_index_map=(0,))
    return lax.gather(v, i[:, None], dimension_numbers=dn, slice_sizes=(1,),
                      mode=lax.GatherScatterMode.PROMISE_IN_BOUNDS)


def _sc_body(sim_hbm, prompt_hbm, key_hbm,
             head_hbm, idx_hbm, bkn_hbm, part_hbm,
             sim_v, idx128_v, idx40_v, rows_v, bkn_v, sum_v, sem, sem2):
    cid = lax.axis_index("c")
    sid = lax.axis_index("s")
    wid = cid * _NS + sid
    lane = lax.iota(jnp.int32, 16)
    neg = jnp.full((16,), -3.0e38, jnp.float32)
    zidx = jnp.zeros((16,), jnp.int32)
    zf = jnp.zeros((16,), jnp.float32)

    sevens = jnp.full((16,), 7, jnp.int32)
    pltpu.sync_copy(sim_hbm.at[pl.ds(wid * _ROWS_PER_W, _ROWS_PER_W)], sim_v)

    def row_body(r, acc):
        b = wid * _ROWS_PER_W + r

        def chunk_body(i, carry):
            bk, bi = carry
            ck = sim_v[r, pl.ds(i * 16, 16)]
            ci = lane + i * 16
            sck, sci = plsc.sort_key_val(ck, ci, descending=True)
            rk = lax.rev(sck, (0,))
            ri = lax.rev(sci, (0,))
            mk = jnp.where(lane < 8, bk, rk)
            mi = jnp.where(lane < 8, bi, ri)
            return tuple(plsc.sort_key_val(mk, mi, descending=True))

        bk, bi = lax.fori_loop(0, _CHUNKS, chunk_body, (neg, zidx))
        iv = jnp.where(lane < 8, bi, jnp.int32(2147483647))
        fi, fv = plsc.sort_key_val(iv, bk, descending=False)
        idx128_v[pl.ds(0, 16)] = fi
        pltpu.sync_copy(idx128_v, idx_hbm.at[b])
        for c in range(3):
            j = lane + 16 * c
            q = lax.div(j, jnp.int32(_L))
            s = j - q * _L
            sel = _vgather16(fi, jnp.minimum(q, jnp.int32(15)))
            idx40_v[pl.ds(16 * c, 16)] = s * _P + sel
        pltpu.async_copy(prompt_hbm.at[idx40_v.at[pl.ds(0, _HEAD)]],
                         rows_v, sem).wait()
        pltpu.sync_copy(rows_v, head_hbm.at[b])
        pltpu.async_copy(key_hbm.at[idx128_v.at[pl.ds(0, 8)]],
                         bkn_v, sem2).wait()
        pltpu.sync_copy(bkn_v, bkn_hbm.at[b])
        sel_sim = jnp.where(lane < 8, fv, jnp.float32(0.0))
        return acc - jnp.sum(sel_sim)

    acc = lax.fori_loop(0, _ROWS_PER_W, row_body, jnp.float32(0.0))
    for c in range(8):
        sum_v[pl.ds(16 * c, 16)] = zf
    sum_v[pl.ds(0, 16)] = jnp.where(lane == 0, jnp.full((16,), acc), zf)
    pltpu.sync_copy(sum_v, part_hbm.at[wid])


def _run_topk_gather(sim, prompt_flat, yn):
    mesh = plsc.VectorSubcoreMesh(core_axis_name="c", subcore_axis_name="s",
                                  num_cores=_NC, num_subcores=_NS)
    fn = pl.kernel(
        _sc_body,
        out_type=[
            jax.ShapeDtypeStruct((_B, _HEAD, _D), jnp.float32),
            jax.ShapeDtypeStruct((_B, 128), jnp.int32),
            jax.ShapeDtypeStruct((_B, _K, _D), jnp.float32),
            jax.ShapeDtypeStruct((_NW, 128), jnp.float32),
        ],
        mesh=mesh,
        compiler_params=pltpu.CompilerParams(needs_layout_passes=False,
                                             use_tc_tiling_on_sc=True),
        scratch_types=[
            pltpu.VMEM((_ROWS_PER_W, _P), jnp.float32),
            pltpu.VMEM((128,), jnp.int32),
            pltpu.VMEM((48,), jnp.int32),
            pltpu.VMEM((_HEAD, _D), jnp.float32),
            pltpu.VMEM((_K, _D), jnp.float32),
            pltpu.VMEM((128,), jnp.float32),
            pltpu.SemaphoreType.DMA,
            pltpu.SemaphoreType.DMA,
        ],
    )
    return fn(sim, prompt_flat, yn)


_DBT = 16


def _assemble_body(big_in_ref, head_ref, part_ref, big_ref, rs_ref):
    del big_in_ref
    big_ref[...] = jnp.transpose(head_ref[...], (1, 0, 2))

    @pl.when(pl.program_id(0) == 0)
    def _():
        rs_ref[...] = jnp.sum(part_ref[...]).reshape(1, 1) / float(_B)


def _run_assemble(bigT0, head, part):
    return pl.pallas_call(
        _assemble_body,
        grid=(_B // _DBT,),
        in_specs=[
            pl.BlockSpec(memory_space=pl.ANY),
            pl.BlockSpec((_DBT, _HEAD, _D), lambda b: (b, 0, 0)),
            pl.BlockSpec((_NW, 128), lambda b: (0, 0)),
        ],
        out_specs=[
            pl.BlockSpec((_HEAD, _DBT, _D), lambda b: (0, b, 0)),
            pl.BlockSpec((1, 1), lambda b: (0, 0)),
        ],
        out_shape=[
            jax.ShapeDtypeStruct((_OUT_S, _B, _D), jnp.float32),
            jax.ShapeDtypeStruct((1, 1), jnp.float32),
        ],
        input_output_aliases={0: 0},
    )(bigT0, head, part)


def kernel(x_embed, prompt, prompt_key):
    xT = jnp.transpose(x_embed, (1, 0, 2))
    bigT0, sim, yn = _run_mean_copy_dist(xT, prompt_key)
    prompt_flat = jnp.transpose(prompt, (1, 0, 2)).reshape(_L * _P, _D)
    head, idx_pad, bkn, part = _run_topk_gather(sim, prompt_flat, yn)
    bigT, rs = _run_assemble(bigT0, head, part)
    big = jnp.transpose(bigT, (1, 0, 2))
    return big, rs[0, 0], sim, idx_pad[:, :_K], bkn


# speedup vs baseline: 1.1155x; 1.0589x over previous
import functools

import jax
import jax.numpy as jnp
from jax import lax
from jax.experimental import pallas as pl
from jax.experimental.pallas import tpu as pltpu
from jax.experimental.pallas import tpu_sc as plsc

_SCALE = 0.1
_K = 8
_L = 5
_P = 1024
_D = 768
_B = 128
_S = 196
_OUT_S = _K * _L + _S
_HEAD = _K * _L
_ROW_W = _L * _D

_NC = 2
_NS = 16
_NW = _NC * _NS
_ROWS_PER_W = _B // _NW
_CHUNKS = _P // 16


_ST = 4


def _map_to_ball(u):
    ss = jnp.sum(u * u, axis=-1, keepdims=True)
    un = u * lax.rsqrt(jnp.maximum(ss, 1e-12))
    us = un * _SCALE
    n2 = jnp.sum(us * us, axis=-1, keepdims=True)
    n = jnp.sqrt(jnp.maximum(n2, 1e-15))
    y = jnp.tanh(n) * us / n
    yn2 = jnp.sum(y * y, axis=-1, keepdims=True)
    ynorm = jnp.sqrt(jnp.maximum(yn2, 1e-15))
    maxnorm = 1.0 - 4e-3
    return jnp.where(ynorm > maxnorm, y / ynorm * maxnorm, y)


def _mean_copy_dist_body(x_ref, pk_ref, big_ref, sim_ref, yn_ref, sum_ref):
    j = pl.program_id(0)
    xb = x_ref[...]
    big_ref[...] = xb

    @pl.when(j == 0)
    def _():
        sum_ref[...] = jnp.zeros((_B, _D), jnp.float32)

    sum_ref[...] += jnp.sum(xb, axis=0)

    @pl.when(j == _S // _ST - 1)
    def _():
        x = _map_to_ball(sum_ref[...] / float(_S))
        y = _map_to_ball(pk_ref[...])
        yn_ref[...] = y
        x2 = jnp.sum(x * x, axis=-1, keepdims=True)
        y2 = jnp.sum(y * y, axis=-1)[None, :]
        xy = lax.dot_general(x, y, (((1,), (1,)), ((), ())),
                             precision=lax.Precision.HIGHEST,
                             preferred_element_type=jnp.float32)
        alpha = 1.0 - 2.0 * xy + y2
        beta = 1.0 - x2
        num2 = alpha * alpha * x2 + beta * beta * y2 - 2.0 * alpha * beta * xy
        den = 1.0 - 2.0 * xy + x2 * y2
        norm = jnp.sqrt(jnp.maximum(num2, 1e-15)) / (den + 1e-15)
        arg = jnp.clip(norm, 0.0, 1.0 - 1e-7)
        dist = jnp.log((1.0 + arg) / (1.0 - arg))
        sim_ref[...] = -dist


def _run_mean_copy_dist(xT, prompt_key):
    return pl.pallas_call(
        _mean_copy_dist_body,
        grid=(_S // _ST,),
        in_specs=[
            pl.BlockSpec((_ST, _B, _D), lambda j: (j, 0, 0)),
            pl.BlockSpec((_P, _D), lambda j: (0, 0)),
        ],
        out_specs=[
            pl.BlockSpec((_ST, _B, _D), lambda j: (j + _HEAD // _ST, 0, 0)),
            pl.BlockSpec((_B, _P), lambda j: (0, 0)),
            pl.BlockSpec((_P, _D), lambda j: (0, 0)),
        ],
        out_shape=[
            jax.ShapeDtypeStruct((_OUT_S, _B, _D), jnp.float32),
            jax.ShapeDtypeStruct((_B, _P), jnp.float32),
            jax.ShapeDtypeStruct((_P, _D), jnp.float32),
        ],
        scratch_shapes=[pltpu.VMEM((_B, _D), jnp.float32)],
    )(xT, prompt_key)


def _vgather16(v, i):
    dn = lax.GatherDimensionNumbers(offset_dims=(), collapsed_slice_dims=(0,),
                                    start_index_map=(0,))
    return lax.gather(v, i[:, None], dimension_numbers=dn, slice_sizes=(1,),
                      mode=lax.GatherScatterMode.PROMISE_IN_BOUNDS)


def _sc_body(sim_hbm, prompt_hbm, key_hbm,
             head_hbm, idx_hbm, bkn_hbm, part_hbm,
             sim_v, idx128s, idx40s, rows2, bkn4, sum_v,
             sP0, sP1, sH0, sH1, sK, sB, sI):
    cid = lax.axis_index("c")
    sid = lax.axis_index("s")
    wid = cid * _NS + sid
    lane = lax.iota(jnp.int32, 16)
    neg = jnp.full((16,), -3.0e38, jnp.float32)
    zidx = jnp.zeros((16,), jnp.int32)
    zf = jnp.zeros((16,), jnp.float32)
    base_b = wid * _ROWS_PER_W

    pltpu.sync_copy(sim_hbm.at[pl.ds(base_b, _ROWS_PER_W)], sim_v)

    sP = (sP0, sP1)
    sH = (sH0, sH1)
    nr = _ROWS_PER_W
    hP = [None] * nr
    hH = [None] * nr
    hK = [None] * nr
    hI = [None] * nr

    acc = jnp.float32(0.0)
    for r in range(nr):
        slot = r % 2

        def chunk_body(i, carry, r=r):
            bk, bi = carry
            ck = sim_v[r, pl.ds(i * 16, 16)]
            ci = lane + i * 16
            sck, sci = plsc.sort_key_val(ck, ci, descending=True)
            rk = lax.rev(sck, (0,))
            ri = lax.rev(sci, (0,))
            mk = jnp.where(lane < 8, bk, rk)
            mi = jnp.where(lane < 8, bi, ri)
            return tuple(plsc.sort_key_val(mk, mi, descending=True))

        bk, bi = lax.fori_loop(0, _CHUNKS, chunk_body, (neg, zidx))
        iv = jnp.where(lane < 8, bi, jnp.int32(2147483647))
        fi, fv = plsc.sort_key_val(iv, bk, descending=False)
        acc = acc - jnp.sum(jnp.where(lane < 8, fv, jnp.float32(0.0)))

        if r >= 2:
            hH[r - 2].wait()

        idx128s[r, pl.ds(0, 16)] = fi
        for c in range(3):
            j = lane + 16 * c
            q = lax.div(j, jnp.int32(_L))
            s = j - q * _L
            sel = _vgather16(fi, jnp.minimum(q, jnp.int32(15)))
            idx40s[r, pl.ds(16 * c, 16)] = s * _P + sel

        hI[r] = pltpu.async_copy(idx128s.at[r], idx_hbm.at[base_b + r], sI)
        hP[r] = pltpu.async_copy(prompt_hbm.at[idx40s.at[r, pl.ds(0, _HEAD)]],
                                 rows2.at[slot], sP[slot])
        hK[r] = pltpu.async_copy(key_hbm.at[idx128s.at[r, pl.ds(0, 8)]],
                                 bkn4.at[r], sK)
        if r >= 1:
            hP[r - 1].wait()
            hH[r - 1] = pltpu.async_copy(
                rows2.at[1 - slot], head_hbm.at[base_b + r - 1], sH[1 - slot])

    hP[nr - 1].wait()
    hH[nr - 1] = pltpu.async_copy(
        rows2.at[(nr - 1) % 2], head_hbm.at[base_b + nr - 1], sH[(nr - 1) % 2])
    for r in range(nr):
        hK[r].wait()
    hB = [None] * nr
    for r in range(nr):
        hB[r] = pltpu.async_copy(bkn4.at[r], bkn_hbm.at[base_b + r], sB)
    hH[nr - 2].wait()
    hH[nr - 1].wait()
    for r in range(nr):
        hB[r].wait()
        hI[r].wait()

    for c in range(8):
        sum_v[pl.ds(16 * c, 16)] = zf
    sum_v[pl.ds(0, 16)] = jnp.where(lane == 0, jnp.full((16,), acc), zf)
    pltpu.sync_copy(sum_v, part_hbm.at[wid])


def _run_topk_gather(sim, prompt_flat, yn):
    mesh = plsc.VectorSubcoreMesh(core_axis_name="c", subcore_axis_name="s",
                                  num_cores=_NC, num_subcores=_NS)
    fn = pl.kernel(
        _sc_body,
        out_type=[
            jax.ShapeDtypeStruct((_B, _HEAD, _D), jnp.float32),
            jax.ShapeDtypeStruct((_B, 128), jnp.int32),
            jax.ShapeDtypeStruct((_B, _K, _D), jnp.float32),
            jax.ShapeDtypeStruct((_NW, 128), jnp.float32),
        ],
        mesh=mesh,
        compiler_params=pltpu.CompilerParams(needs_layout_passes=False,
                                             use_tc_tiling_on_sc=True),
        scratch_types=[
            pltpu.VMEM((_ROWS_PER_W, _P), jnp.float32),
            pltpu.VMEM((_ROWS_PER_W, 128), jnp.int32),
            pltpu.VMEM((_ROWS_PER_W, 48), jnp.int32),
            pltpu.VMEM((2, _HEAD, _D), jnp.float32),
            pltpu.VMEM((_ROWS_PER_W, _K, _D), jnp.float32),
            pltpu.VMEM((128,), jnp.float32),
            pltpu.SemaphoreType.DMA,
            pltpu.SemaphoreType.DMA,
            pltpu.SemaphoreType.DMA,
            pltpu.SemaphoreType.DMA,
            pltpu.SemaphoreType.DMA,
            pltpu.SemaphoreType.DMA,
            pltpu.SemaphoreType.DMA,
        ],
    )
    return fn(sim, prompt_flat, yn)


_DBT = 16


def _assemble_body(big_in_ref, head_ref, part_ref, big_ref, rs_ref):
    del big_in_ref
    big_ref[...] = jnp.transpose(head_ref[...], (1, 0, 2))

    @pl.when(pl.program_id(0) == 0)
    def _():
        rs_ref[...] = jnp.sum(part_ref[...]).reshape(1, 1) / float(_B)


def _run_assemble(bigT0, head, part):
    return pl.pallas_call(
        _assemble_body,
        grid=(_B // _DBT,),
        in_specs=[
            pl.BlockSpec(memory_space=pl.ANY),
            pl.BlockSpec((_DBT, _HEAD, _D), lambda b: (b, 0, 0)),
            pl.BlockSpec((_NW, 128), lambda b: (0, 0)),
        ],
        out_specs=[
            pl.BlockSpec((_HEAD, _DBT, _D), lambda b: (0, b, 0)),
            pl.BlockSpec((1, 1), lambda b: (0, 0)),
        ],
        out_shape=[
            jax.ShapeDtypeStruct((_OUT_S, _B, _D), jnp.float32),
            jax.ShapeDtypeStruct((1, 1), jnp.float32),
        ],
        input_output_aliases={0: 0},
    )(bigT0, head, part)


def kernel(x_embed, prompt, prompt_key):
    xT = jnp.transpose(x_embed, (1, 0, 2))
    bigT0, sim, yn = _run_mean_copy_dist(xT, prompt_key)
    prompt_flat = jnp.transpose(prompt, (1, 0, 2)).reshape(_L * _P, _D)
    head, idx_pad, bkn, part = _run_topk_gather(sim, prompt_flat, yn)
    bigT, rs = _run_assemble(bigT0, head, part)
    big = jnp.transpose(bigT, (1, 0, 2))
    return big, rs[0, 0], sim, idx_pad[:, :_K], bkn


# speedup vs baseline: 1.1262x; 1.0096x over previous
import functools

import jax
import jax.numpy as jnp
from jax import lax
from jax.experimental import pallas as pl
from jax.experimental.pallas import tpu as pltpu
from jax.experimental.pallas import tpu_sc as plsc

_SCALE = 0.1
_K = 8
_L = 5
_P = 1024
_D = 768
_B = 128
_S = 196
_OUT_S = _K * _L + _S
_HEAD = _K * _L
_ROW_W = _L * _D

_NC = 2
_NS = 16
_NW = _NC * _NS
_ROWS_PER_W = _B // _NW
_CHUNKS = _P // 16


_ST = 4


def _map_to_ball(u):
    ss = jnp.sum(u * u, axis=-1, keepdims=True)
    un = u * lax.rsqrt(jnp.maximum(ss, 1e-12))
    us = un * _SCALE
    n2 = jnp.sum(us * us, axis=-1, keepdims=True)
    n = jnp.sqrt(jnp.maximum(n2, 1e-15))
    y = jnp.tanh(n) * us / n
    yn2 = jnp.sum(y * y, axis=-1, keepdims=True)
    ynorm = jnp.sqrt(jnp.maximum(yn2, 1e-15))
    maxnorm = 1.0 - 4e-3
    return jnp.where(ynorm > maxnorm, y / ynorm * maxnorm, y)


def _mean_copy_dist_body(x_ref, pk_ref, big_ref, sim_ref, yn_ref, sum_ref):
    j = pl.program_id(0)
    xb = x_ref[...]
    big_ref[...] = xb

    @pl.when(j == 0)
    def _():
        sum_ref[...] = jnp.zeros((_B, _D), jnp.float32)

    sum_ref[...] += jnp.sum(xb, axis=0)

    @pl.when(j == _S // _ST - 1)
    def _():
        x = _map_to_ball(sum_ref[...] / float(_S))
        y = _map_to_ball(pk_ref[...])
        yn_ref[...] = y
        x2 = jnp.sum(x * x, axis=-1, keepdims=True)
        y2 = jnp.sum(y * y, axis=-1)[None, :]
        xy = lax.dot_general(x, y, (((1,), (1,)), ((), ())),
                             precision=lax.Precision.HIGHEST,
                             preferred_element_type=jnp.float32)
        alpha = 1.0 - 2.0 * xy + y2
        beta = 1.0 - x2
        num2 = alpha * alpha * x2 + beta * beta * y2 - 2.0 * alpha * beta * xy
        den = 1.0 - 2.0 * xy + x2 * y2
        norm = jnp.sqrt(jnp.maximum(num2, 1e-15)) / (den + 1e-15)
        arg = jnp.clip(norm, 0.0, 1.0 - 1e-7)
        dist = jnp.log((1.0 + arg) / (1.0 - arg))
        sim_ref[...] = -dist


def _run_mean_copy_dist(xT, prompt_key):
    return pl.pallas_call(
        _mean_copy_dist_body,
        grid=(_S // _ST,),
        in_specs=[
            pl.BlockSpec((_ST, _B, _D), lambda j: (j, 0, 0)),
            pl.BlockSpec((_P, _D), lambda j: (0, 0)),
        ],
        out_specs=[
            pl.BlockSpec((_ST, _B, _D), lambda j: (j + _HEAD // _ST, 0, 0)),
            pl.BlockSpec((_B, _P), lambda j: (0, 0)),
            pl.BlockSpec((_P, _D), lambda j: (0, 0)),
        ],
        out_shape=[
            jax.ShapeDtypeStruct((_OUT_S, _B, _D), jnp.float32),
            jax.ShapeDtypeStruct((_B, _P), jnp.float32),
            jax.ShapeDtypeStruct((_P, _D), jnp.float32),
        ],
        scratch_shapes=[pltpu.VMEM((_B, _D), jnp.float32)],
    )(xT, prompt_key)


def _vgather16(v, i):
    dn = lax.GatherDimensionNumbers(offset_dims=(), collapsed_slice_dims=(0,),
                                    start_index_map=(0,))
    return lax.gather(v, i[:, None], dimension_numbers=dn, slice_sizes=(1,),
                      mode=lax.GatherScatterMode.PROMISE_IN_BOUNDS)


def _sc_body(sim_hbm, prompt_hbm, key_hbm,
             head_hbm, idx_hbm, bkn_hbm, part_hbm,
             sim_v, idx128s, idx40s, rows2, bkn4, sum_v,
             sP0, sP1, sH0, sH1, sK, sB, sI):
    cid = lax.axis_index("c")
    sid = lax.axis_index("s")
    wid = cid * _NS + sid
    lane = lax.iota(jnp.int32, 16)
    neg = jnp.full((16,), -3.0e38, jnp.float32)
    zidx = jnp.zeros((16,), jnp.int32)
    zf = jnp.zeros((16,), jnp.float32)
    base_b = wid * _ROWS_PER_W

    pltpu.sync_copy(sim_hbm.at[pl.ds(base_b, _ROWS_PER_W)], sim_v)

    sP = (sP0, sP1)
    sH = (sH0, sH1)
    nr = _ROWS_PER_W
    hP = [None] * nr
    hH = [None] * nr
    hK = [None] * nr
    hI = [None] * nr

    acc = jnp.float32(0.0)
    for r in range(nr):
        slot = r % 2

        def chunk_body(i, carry, r=r):
            bk, bi = carry
            cka = sim_v[r, pl.ds(i * 32, 16)]
            ckb = sim_v[r, pl.ds(i * 32 + 16, 16)]
            cia = lane + i * 32
            cib = cia + 16
            ska, sia = plsc.sort_key_val(cka, cia, descending=True)
            skb, sib = plsc.sort_key_val(ckb, cib, descending=True)
            pk_ = jnp.where(lane < 8, ska, lax.rev(skb, (0,)))
            pi_ = jnp.where(lane < 8, sia, lax.rev(sib, (0,)))
            spk, spi = plsc.sort_key_val(pk_, pi_, descending=True)
            mk = jnp.where(lane < 8, bk, lax.rev(spk, (0,)))
            mi = jnp.where(lane < 8, bi, lax.rev(spi, (0,)))
            return tuple(plsc.sort_key_val(mk, mi, descending=True))

        bk, bi = lax.fori_loop(0, _CHUNKS // 2, chunk_body, (neg, zidx))
        iv = jnp.where(lane < 8, bi, jnp.int32(2147483647))
        fi, fv = plsc.sort_key_val(iv, bk, descending=False)
        acc = acc - jnp.sum(jnp.where(lane < 8, fv, jnp.float32(0.0)))

        if r >= 2:
            hH[r - 2].wait()

        idx128s[r, pl.ds(0, 16)] = fi
        for c in range(3):
            j = lane + 16 * c
            q = lax.div(j, jnp.int32(_L))
            s = j - q * _L
            sel = _vgather16(fi, jnp.minimum(q, jnp.int32(15)))
            idx40s[r, pl.ds(16 * c, 16)] = s * _P + sel

        hI[r] = pltpu.async_copy(idx128s.at[r], idx_hbm.at[base_b + r], sI)
        hP[r] = pltpu.async_copy(prompt_hbm.at[idx40s.at[r, pl.ds(0, _HEAD)]],
                                 rows2.at[slot], sP[slot])
        hK[r] = pltpu.async_copy(key_hbm.at[idx128s.at[r, pl.ds(0, 8)]],
                                 bkn4.at[r], sK)
        if r >= 1:
            hP[r - 1].wait()
            hH[r - 1] = pltpu.async_copy(
                rows2.at[1 - slot], head_hbm.at[base_b + r - 1], sH[1 - slot])

    hP[nr - 1].wait()
    hH[nr - 1] = pltpu.async_copy(
        rows2.at[(nr - 1) % 2], head_hbm.at[base_b + nr - 1], sH[(nr - 1) % 2])
    for r in range(nr):
        hK[r].wait()
    hB = [None] * nr
    for r in range(nr):
        hB[r] = pltpu.async_copy(bkn4.at[r], bkn_hbm.at[base_b + r], sB)
    hH[nr - 2].wait()
    hH[nr - 1].wait()
    for r in range(nr):
        hB[r].wait()
        hI[r].wait()

    for c in range(8):
        sum_v[pl.ds(16 * c, 16)] = zf
    sum_v[pl.ds(0, 16)] = jnp.where(lane == 0, jnp.full((16,), acc), zf)
    pltpu.sync_copy(sum_v, part_hbm.at[wid])


def _run_topk_gather(sim, prompt_flat, yn):
    mesh = plsc.VectorSubcoreMesh(core_axis_name="c", subcore_axis_name="s",
                                  num_cores=_NC, num_subcores=_NS)
    fn = pl.kernel(
        _sc_body,
        out_type=[
            jax.ShapeDtypeStruct((_B, _HEAD, _D), jnp.float32),
            jax.ShapeDtypeStruct((_B, 128), jnp.int32),
            jax.ShapeDtypeStruct((_B, _K, _D), jnp.float32),
            jax.ShapeDtypeStruct((_NW, 128), jnp.float32),
        ],
        mesh=mesh,
        compiler_params=pltpu.CompilerParams(needs_layout_passes=False,
                                             use_tc_tiling_on_sc=True),
        scratch_types=[
            pltpu.VMEM((_ROWS_PER_W, _P), jnp.float32),
            pltpu.VMEM((_ROWS_PER_W, 128), jnp.int32),
            pltpu.VMEM((_ROWS_PER_W, 48), jnp.int32),
            pltpu.VMEM((2, _HEAD, _D), jnp.float32),
            pltpu.VMEM((_ROWS_PER_W, _K, _D), jnp.float32),
            pltpu.VMEM((128,), jnp.float32),
            pltpu.SemaphoreType.DMA,
            pltpu.SemaphoreType.DMA,
            pltpu.SemaphoreType.DMA,
            pltpu.SemaphoreType.DMA,
            pltpu.SemaphoreType.DMA,
            pltpu.SemaphoreType.DMA,
            pltpu.SemaphoreType.DMA,
        ],
    )
    return fn(sim, prompt_flat, yn)


_DBT = 32


def _assemble_body(big_in_ref, head_ref, part_ref, big_ref, rs_ref):
    del big_in_ref
    big_ref[...] = jnp.transpose(head_ref[...], (1, 0, 2))

    @pl.when(pl.program_id(0) == 0)
    def _():
        rs_ref[...] = jnp.sum(part_ref[...]).reshape(1, 1) / float(_B)


def _run_assemble(bigT0, head, part):
    return pl.pallas_call(
        _assemble_body,
        grid=(_B // _DBT,),
        in_specs=[
            pl.BlockSpec(memory_space=pl.ANY),
            pl.BlockSpec((_DBT, _HEAD, _D), lambda b: (b, 0, 0)),
            pl.BlockSpec((_NW, 128), lambda b: (0, 0)),
        ],
        out_specs=[
            pl.BlockSpec((_HEAD, _DBT, _D), lambda b: (0, b, 0)),
            pl.BlockSpec((1, 1), lambda b: (0, 0)),
        ],
        out_shape=[
            jax.ShapeDtypeStruct((_OUT_S, _B, _D), jnp.float32),
            jax.ShapeDtypeStruct((1, 1), jnp.float32),
        ],
        input_output_aliases={0: 0},
    )(bigT0, head, part)


def kernel(x_embed, prompt, prompt_key):
    xT = jnp.transpose(x_embed, (1, 0, 2))
    bigT0, sim, yn = _run_mean_copy_dist(xT, prompt_key)
    prompt_flat = jnp.transpose(prompt, (1, 0, 2)).reshape(_L * _P, _D)
    head, idx_pad, bkn, part = _run_topk_gather(sim, prompt_flat, yn)
    bigT, rs = _run_assemble(bigT0, head, part)
    big = jnp.transpose(bigT, (1, 0, 2))
    return big, rs[0, 0], sim, idx_pad[:, :_K], bkn


# speedup vs baseline: 1.2629x; 1.1215x over previous
import functools

import jax
import jax.numpy as jnp
from jax import lax
from jax.experimental import pallas as pl
from jax.experimental.pallas import tpu as pltpu
from jax.experimental.pallas import tpu_sc as plsc

_SCALE = 0.1
_K = 8
_L = 5
_P = 1024
_D = 768
_B = 128
_S = 196
_OUT_S = _K * _L + _S
_HEAD = _K * _L
_ROW_W = _L * _D

_NC = 2
_NS = 16
_NW = _NC * _NS
_ROWS_PER_W = _B // _NW
_CHUNKS = _P // 16


_ST = 4


def _map_to_ball(u):
    ss = jnp.sum(u * u, axis=-1, keepdims=True)
    un = u * lax.rsqrt(jnp.maximum(ss, 1e-12))
    us = un * _SCALE
    n2 = jnp.sum(us * us, axis=-1, keepdims=True)
    n = jnp.sqrt(jnp.maximum(n2, 1e-15))
    y = jnp.tanh(n) * us / n
    yn2 = jnp.sum(y * y, axis=-1, keepdims=True)
    ynorm = jnp.sqrt(jnp.maximum(yn2, 1e-15))
    maxnorm = 1.0 - 4e-3
    return jnp.where(ynorm > maxnorm, y / ynorm * maxnorm, y)


_AST = 28
_ANS = _S // _AST


def _mean_copy_dist_body(x_any, pk_ref, big_any, sim_ref, yn_ref,
                         xbuf, sum_ref, s_in, s_out):
    j = pl.program_id(0)
    slot = j % 2
    nslot = 1 - slot

    @pl.when(j == 0)
    def _():
        pltpu.make_async_copy(x_any.at[pl.ds(0, _AST)], xbuf.at[0],
                              s_in.at[0]).start()
        sum_ref[...] = jnp.zeros((_B, _D), jnp.float32)

    @pl.when(j >= 1)
    def _():
        pltpu.make_async_copy(
            xbuf.at[nslot],
            big_any.at[pl.ds(_HEAD + (j - 1) * _AST, _AST)],
            s_out.at[nslot]).wait()

    @pl.when(j + 1 < _ANS)
    def _():
        pltpu.make_async_copy(x_any.at[pl.ds((j + 1) * _AST, _AST)],
                              xbuf.at[nslot], s_in.at[nslot]).start()

    pltpu.make_async_copy(x_any.at[pl.ds(j * _AST, _AST)], xbuf.at[slot],
                          s_in.at[slot]).wait()
    pltpu.make_async_copy(xbuf.at[slot],
                          big_any.at[pl.ds(_HEAD + j * _AST, _AST)],
                          s_out.at[slot]).start()

    def acc_row(i, s):
        return s + xbuf[slot, i]

    sum_ref[...] += lax.fori_loop(
        0, _AST, acc_row, jnp.zeros((_B, _D), jnp.float32))

    @pl.when(j == _ANS - 1)
    def _():
        pltpu.make_async_copy(
            xbuf.at[slot],
            big_any.at[pl.ds(_HEAD + j * _AST, _AST)],
            s_out.at[slot]).wait()
        x = _map_to_ball(sum_ref[...] / float(_S))
        y = _map_to_ball(pk_ref[...])
        yn_ref[...] = y
        x2 = jnp.sum(x * x, axis=-1, keepdims=True)
        y2 = jnp.sum(y * y, axis=-1)[None, :]
        xy = lax.dot_general(x, y, (((1,), (1,)), ((), ())),
                             precision=lax.Precision.HIGHEST,
                             preferred_element_type=jnp.float32)
        alpha = 1.0 - 2.0 * xy + y2
        beta = 1.0 - x2
        num2 = alpha * alpha * x2 + beta * beta * y2 - 2.0 * alpha * beta * xy
        den = 1.0 - 2.0 * xy + x2 * y2
        norm = jnp.sqrt(jnp.maximum(num2, 1e-15)) / (den + 1e-15)
        arg = jnp.clip(norm, 0.0, 1.0 - 1e-7)
        dist = jnp.log((1.0 + arg) / (1.0 - arg))
        sim_ref[...] = -dist


def _run_mean_copy_dist(xT, prompt_key):
    return pl.pallas_call(
        _mean_copy_dist_body,
        grid=(_ANS,),
        in_specs=[
            pl.BlockSpec(memory_space=pl.ANY),
            pl.BlockSpec((_P, _D), lambda j: (0, 0)),
        ],
        out_specs=[
            pl.BlockSpec(memory_space=pl.ANY),
            pl.BlockSpec((_B, _P), lambda j: (0, 0)),
            pl.BlockSpec((_P, _D), lambda j: (0, 0)),
        ],
        out_shape=[
            jax.ShapeDtypeStruct((_OUT_S, _B, _D), jnp.float32),
            jax.ShapeDtypeStruct((_B, _P), jnp.float32),
            jax.ShapeDtypeStruct((_P, _D), jnp.float32),
        ],
        scratch_shapes=[
            pltpu.VMEM((2, _AST, _B, _D), jnp.float32),
            pltpu.VMEM((_B, _D), jnp.float32),
            pltpu.SemaphoreType.DMA((2,)),
            pltpu.SemaphoreType.DMA((2,)),
        ],
    )(xT, prompt_key)


def _vgather16(v, i):
    dn = lax.GatherDimensionNumbers(offset_dims=(), collapsed_slice_dims=(0,),
                                    start_index_map=(0,))
    return lax.gather(v, i[:, None], dimension_numbers=dn, slice_sizes=(1,),
                      mode=lax.GatherScatterMode.PROMISE_IN_BOUNDS)


def _sc_body(sim_hbm, prompt_hbm, key_hbm,
             head_hbm, idx_hbm, bkn_hbm, part_hbm,
             sim_v, idx128s, idx40s, rows2, bkn4, sum_v,
             sP0, sP1, sH0, sH1, sK, sB, sI):
    cid = lax.axis_index("c")
    sid = lax.axis_index("s")
    wid = cid * _NS + sid
    lane = lax.iota(jnp.int32, 16)
    neg = jnp.full((16,), -3.0e38, jnp.float32)
    zidx = jnp.zeros((16,), jnp.int32)
    zf = jnp.zeros((16,), jnp.float32)
    base_b = wid * _ROWS_PER_W

    pltpu.sync_copy(sim_hbm.at[pl.ds(base_b, _ROWS_PER_W)], sim_v)

    sP = (sP0, sP1)
    sH = (sH0, sH1)
    nr = _ROWS_PER_W
    hP = [None] * nr
    hH = [None] * nr
    hK = [None] * nr
    hI = [None] * nr

    acc = jnp.float32(0.0)
    for r in range(nr):
        slot = r % 2

        def chunk_body(i, carry, r=r):
            bk, bi = carry
            cka = sim_v[r, pl.ds(i * 32, 16)]
            ckb = sim_v[r, pl.ds(i * 32 + 16, 16)]
            cia = lane + i * 32
            cib = cia + 16
            ska, sia = plsc.sort_key_val(cka, cia, descending=True)
            skb, sib = plsc.sort_key_val(ckb, cib, descending=True)
            pk_ = jnp.where(lane < 8, ska, lax.rev(skb, (0,)))
            pi_ = jnp.where(lane < 8, sia, lax.rev(sib, (0,)))
            spk, spi = plsc.sort_key_val(pk_, pi_, descending=True)
            mk = jnp.where(lane < 8, bk, lax.rev(spk, (0,)))
            mi = jnp.where(lane < 8, bi, lax.rev(spi, (0,)))
            return tuple(plsc.sort_key_val(mk, mi, descending=True))

        bk, bi = lax.fori_loop(0, _CHUNKS // 2, chunk_body, (neg, zidx))
        iv = jnp.where(lane < 8, bi, jnp.int32(2147483647))
        fi, fv = plsc.sort_key_val(iv, bk, descending=False)
        acc = acc - jnp.sum(jnp.where(lane < 8, fv, jnp.float32(0.0)))

        if r >= 2:
            hH[r - 2].wait()

        idx128s[r, pl.ds(0, 16)] = fi
        for c in range(3):
            j = lane + 16 * c
            q = lax.div(j, jnp.int32(_L))
            s = j - q * _L
            sel = _vgather16(fi, jnp.minimum(q, jnp.int32(15)))
            idx40s[r, pl.ds(16 * c, 16)] = s * _P + sel

        hI[r] = pltpu.async_copy(idx128s.at[r], idx_hbm.at[base_b + r], sI)
        hP[r] = pltpu.async_copy(prompt_hbm.at[idx40s.at[r, pl.ds(0, _HEAD)]],
                                 rows2.at[slot], sP[slot])
        hK[r] = pltpu.async_copy(key_hbm.at[idx128s.at[r, pl.ds(0, 8)]],
                                 bkn4.at[r], sK)
        if r >= 1:
            hP[r - 1].wait()
            hH[r - 1] = pltpu.async_copy(
                rows2.at[1 - slot], head_hbm.at[base_b + r - 1], sH[1 - slot])

    hP[nr - 1].wait()
    hH[nr - 1] = pltpu.async_copy(
        rows2.at[(nr - 1) % 2], head_hbm.at[base_b + nr - 1], sH[(nr - 1) % 2])
    for r in range(nr):
        hK[r].wait()
    hB = [None] * nr
    for r in range(nr):
        hB[r] = pltpu.async_copy(bkn4.at[r], bkn_hbm.at[base_b + r], sB)
    hH[nr - 2].wait()
    hH[nr - 1].wait()
    for r in range(nr):
        hB[r].wait()
        hI[r].wait()

    for c in range(8):
        sum_v[pl.ds(16 * c, 16)] = zf
    sum_v[pl.ds(0, 16)] = jnp.where(lane == 0, jnp.full((16,), acc), zf)
    pltpu.sync_copy(sum_v, part_hbm.at[wid])


def _run_topk_gather(sim, prompt_flat, yn):
    mesh = plsc.VectorSubcoreMesh(core_axis_name="c", subcore_axis_name="s",
                                  num_cores=_NC, num_subcores=_NS)
    fn = pl.kernel(
        _sc_body,
        out_type=[
            jax.ShapeDtypeStruct((_B, _HEAD, _D), jnp.float32),
            jax.ShapeDtypeStruct((_B, 128), jnp.int32),
            jax.ShapeDtypeStruct((_B, _K, _D), jnp.float32),
            jax.ShapeDtypeStruct((_NW, 128), jnp.float32),
        ],
        mesh=mesh,
        compiler_params=pltpu.CompilerParams(needs_layout_passes=False,
                                             use_tc_tiling_on_sc=True),
        scratch_types=[
            pltpu.VMEM((_ROWS_PER_W, _P), jnp.float32),
            pltpu.VMEM((_ROWS_PER_W, 128), jnp.int32),
            pltpu.VMEM((_ROWS_PER_W, 48), jnp.int32),
            pltpu.VMEM((2, _HEAD, _D), jnp.float32),
            pltpu.VMEM((_ROWS_PER_W, _K, _D), jnp.float32),
            pltpu.VMEM((128,), jnp.float32),
            pltpu.SemaphoreType.DMA,
            pltpu.SemaphoreType.DMA,
            pltpu.SemaphoreType.DMA,
            pltpu.SemaphoreType.DMA,
            pltpu.SemaphoreType.DMA,
            pltpu.SemaphoreType.DMA,
            pltpu.SemaphoreType.DMA,
        ],
    )
    return fn(sim, prompt_flat, yn)


_DBT = 32


def _assemble_body(big_in_ref, head_ref, part_ref, big_ref, rs_ref):
    del big_in_ref
    big_ref[...] = jnp.transpose(head_ref[...], (1, 0, 2))

    @pl.when(pl.program_id(0) == 0)
    def _():
        rs_ref[...] = jnp.sum(part_ref[...]).reshape(1, 1) / float(_B)


def _run_assemble(bigT0, head, part):
    return pl.pallas_call(
        _assemble_body,
        grid=(_B // _DBT,),
        in_specs=[
            pl.BlockSpec(memory_space=pl.ANY),
            pl.BlockSpec((_DBT, _HEAD, _D), lambda b: (b, 0, 0)),
            pl.BlockSpec((_NW, 128), lambda b: (0, 0)),
        ],
        out_specs=[
            pl.BlockSpec((_HEAD, _DBT, _D), lambda b: (0, b, 0)),
            pl.BlockSpec((1, 1), lambda b: (0, 0)),
        ],
        out_shape=[
            jax.ShapeDtypeStruct((_OUT_S, _B, _D), jnp.float32),
            jax.ShapeDtypeStruct((1, 1), jnp.float32),
        ],
        input_output_aliases={0: 0},
    )(bigT0, head, part)


def kernel(x_embed, prompt, prompt_key):
    xT = jnp.transpose(x_embed, (1, 0, 2))
    bigT0, sim, yn = _run_mean_copy_dist(xT, prompt_key)
    prompt_flat = jnp.transpose(prompt, (1, 0, 2)).reshape(_L * _P, _D)
    head, idx_pad, bkn, part = _run_topk_gather(sim, prompt_flat, yn)
    bigT, rs = _run_assemble(bigT0, head, part)
    big = jnp.transpose(bigT, (1, 0, 2))
    return big, rs[0, 0], sim, idx_pad[:, :_K], bkn


# speedup vs baseline: 1.2744x; 1.0091x over previous
import functools

import jax
import jax.numpy as jnp
from jax import lax
from jax.experimental import pallas as pl
from jax.experimental.pallas import tpu as pltpu
from jax.experimental.pallas import tpu_sc as plsc

_SCALE = 0.1
_K = 8
_L = 5
_P = 1024
_D = 768
_B = 128
_S = 196
_OUT_S = _K * _L + _S
_HEAD = _K * _L
_ROW_W = _L * _D

_NC = 2
_NS = 16
_NW = _NC * _NS
_ROWS_PER_W = _B // _NW
_CHUNKS = _P // 16


_ST = 4


def _map_to_ball(u):
    ss = jnp.sum(u * u, axis=-1, keepdims=True)
    un = u * lax.rsqrt(jnp.maximum(ss, 1e-12))
    us = un * _SCALE
    n2 = jnp.sum(us * us, axis=-1, keepdims=True)
    n = jnp.sqrt(jnp.maximum(n2, 1e-15))
    y = jnp.tanh(n) * us / n
    yn2 = jnp.sum(y * y, axis=-1, keepdims=True)
    ynorm = jnp.sqrt(jnp.maximum(yn2, 1e-15))
    maxnorm = 1.0 - 4e-3
    return jnp.where(ynorm > maxnorm, y / ynorm * maxnorm, y)


_AST = 49
_ANS = _S // _AST


def _mean_copy_dist_body(x_any, pk_ref, big_any, sim_ref, yn_ref,
                         xbuf, sum_ref, s_in, s_out):
    j = pl.program_id(0)
    slot = j % 2
    nslot = 1 - slot

    @pl.when(j == 0)
    def _():
        pltpu.make_async_copy(x_any.at[pl.ds(0, _AST)], xbuf.at[0],
                              s_in.at[0]).start()
        sum_ref[...] = jnp.zeros((_B, _D), jnp.float32)

    @pl.when(j >= 1)
    def _():
        pltpu.make_async_copy(
            xbuf.at[nslot],
            big_any.at[pl.ds(_HEAD + (j - 1) * _AST, _AST)],
            s_out.at[nslot]).wait()

    @pl.when(j + 1 < _ANS)
    def _():
        pltpu.make_async_copy(x_any.at[pl.ds((j + 1) * _AST, _AST)],
                              xbuf.at[nslot], s_in.at[nslot]).start()

    pltpu.make_async_copy(x_any.at[pl.ds(j * _AST, _AST)], xbuf.at[slot],
                          s_in.at[slot]).wait()
    pltpu.make_async_copy(xbuf.at[slot],
                          big_any.at[pl.ds(_HEAD + j * _AST, _AST)],
                          s_out.at[slot]).start()

    def acc_row(i, s):
        return s + xbuf[slot, i]

    sum_ref[...] += lax.fori_loop(
        0, _AST, acc_row, jnp.zeros((_B, _D), jnp.float32))

    @pl.when(j == _ANS - 1)
    def _():
        pltpu.make_async_copy(
            xbuf.at[slot],
            big_any.at[pl.ds(_HEAD + j * _AST, _AST)],
            s_out.at[slot]).wait()
        x = _map_to_ball(sum_ref[...] / float(_S))
        y = _map_to_ball(pk_ref[...])
        yn_ref[...] = y
        x2 = jnp.sum(x * x, axis=-1, keepdims=True)
        y2 = jnp.sum(y * y, axis=-1)[None, :]
        xy = lax.dot_general(x, y, (((1,), (1,)), ((), ())),
                             precision=lax.Precision.HIGHEST,
                             preferred_element_type=jnp.float32)
        alpha = 1.0 - 2.0 * xy + y2
        beta = 1.0 - x2
        num2 = alpha * alpha * x2 + beta * beta * y2 - 2.0 * alpha * beta * xy
        den = 1.0 - 2.0 * xy + x2 * y2
        norm = jnp.sqrt(jnp.maximum(num2, 1e-15)) / (den + 1e-15)
        arg = jnp.clip(norm, 0.0, 1.0 - 1e-7)
        dist = jnp.log((1.0 + arg) / (1.0 - arg))
        sim_ref[...] = -dist


def _run_mean_copy_dist(xT, prompt_key):
    return pl.pallas_call(
        _mean_copy_dist_body,
        grid=(_ANS,),
        in_specs=[
            pl.BlockSpec(memory_space=pl.ANY),
            pl.BlockSpec((_P, _D), lambda j: (0, 0)),
        ],
        out_specs=[
            pl.BlockSpec(memory_space=pl.ANY),
            pl.BlockSpec((_B, _P), lambda j: (0, 0)),
            pl.BlockSpec((_P, _D), lambda j: (0, 0)),
        ],
        out_shape=[
            jax.ShapeDtypeStruct((_OUT_S, _B, _D), jnp.float32),
            jax.ShapeDtypeStruct((_B, _P), jnp.float32),
            jax.ShapeDtypeStruct((_P, _D), jnp.float32),
        ],
        scratch_shapes=[
            pltpu.VMEM((2, _AST, _B, _D), jnp.float32),
            pltpu.VMEM((_B, _D), jnp.float32),
            pltpu.SemaphoreType.DMA((2,)),
            pltpu.SemaphoreType.DMA((2,)),
        ],
    )(xT, prompt_key)


def _vgather16(v, i):
    dn = lax.GatherDimensionNumbers(offset_dims=(), collapsed_slice_dims=(0,),
                                    start_index_map=(0,))
    return lax.gather(v, i[:, None], dimension_numbers=dn, slice_sizes=(1,),
                      mode=lax.GatherScatterMode.PROMISE_IN_BOUNDS)


def _sc_body(sim_hbm, prompt_hbm, key_hbm,
             head_hbm, idx_hbm, bkn_hbm, part_hbm,
             sim_v, idx128s, idx40s, rows2, bkn4, sum_v,
             sP0, sP1, sH0, sH1, sK, sB, sI):
    cid = lax.axis_index("c")
    sid = lax.axis_index("s")
    wid = cid * _NS + sid
    lane = lax.iota(jnp.int32, 16)
    neg = jnp.full((16,), -3.0e38, jnp.float32)
    zidx = jnp.zeros((16,), jnp.int32)
    zf = jnp.zeros((16,), jnp.float32)
    base_b = wid * _ROWS_PER_W

    pltpu.sync_copy(sim_hbm.at[pl.ds(base_b, _ROWS_PER_W)], sim_v)

    sP = (sP0, sP1)
    sH = (sH0, sH1)
    nr = _ROWS_PER_W
    hP = [None] * nr
    hH = [None] * nr
    hK = [None] * nr
    hI = [None] * nr

    acc = jnp.float32(0.0)
    for r in range(nr):
        slot = r % 2

        def chunk_body(i, carry, r=r):
            bk, bi = carry
            cka = sim_v[r, pl.ds(i * 32, 16)]
            ckb = sim_v[r, pl.ds(i * 32 + 16, 16)]
            cia = lane + i * 32
            cib = cia + 16
            ska, sia = plsc.sort_key_val(cka, cia, descending=True)
            skb, sib = plsc.sort_key_val(ckb, cib, descending=True)
            pk_ = jnp.where(lane < 8, ska, lax.rev(skb, (0,)))
            pi_ = jnp.where(lane < 8, sia, lax.rev(sib, (0,)))
            spk, spi = plsc.sort_key_val(pk_, pi_, descending=True)
            mk = jnp.where(lane < 8, bk, lax.rev(spk, (0,)))
            mi = jnp.where(lane < 8, bi, lax.rev(spi, (0,)))
            return tuple(plsc.sort_key_val(mk, mi, descending=True))

        bk, bi = lax.fori_loop(0, _CHUNKS // 2, chunk_body, (neg, zidx))
        iv = jnp.where(lane < 8, bi, jnp.int32(2147483647))
        fi, fv = plsc.sort_key_val(iv, bk, descending=False)
        acc = acc - jnp.sum(jnp.where(lane < 8, fv, jnp.float32(0.0)))

        if r >= 2:
            hH[r - 2].wait()

        idx128s[r, pl.ds(0, 16)] = fi
        for c in range(3):
            j = lane + 16 * c
            q = lax.div(j, jnp.int32(_L))
            s = j - q * _L
            sel = _vgather16(fi, jnp.minimum(q, jnp.int32(15)))
            idx40s[r, pl.ds(16 * c, 16)] = s * _P + sel

        hI[r] = pltpu.async_copy(idx128s.at[r], idx_hbm.at[base_b + r], sI)
        hP[r] = pltpu.async_copy(prompt_hbm.at[idx40s.at[r, pl.ds(0, _HEAD)]],
                                 rows2.at[slot], sP[slot])
        hK[r] = pltpu.async_copy(key_hbm.at[idx128s.at[r, pl.ds(0, 8)]],
                                 bkn4.at[r], sK)
        if r >= 1:
            hP[r - 1].wait()
            hH[r - 1] = pltpu.async_copy(
                rows2.at[1 - slot], head_hbm.at[base_b + r - 1], sH[1 - slot])

    hP[nr - 1].wait()
    hH[nr - 1] = pltpu.async_copy(
        rows2.at[(nr - 1) % 2], head_hbm.at[base_b + nr - 1], sH[(nr - 1) % 2])
    for r in range(nr):
        hK[r].wait()
    hB = [None] * nr
    for r in range(nr):
        hB[r] = pltpu.async_copy(bkn4.at[r], bkn_hbm.at[base_b + r], sB)
    hH[nr - 2].wait()
    hH[nr - 1].wait()
    for r in range(nr):
        hB[r].wait()
        hI[r].wait()

    for c in range(8):
        sum_v[pl.ds(16 * c, 16)] = zf
    sum_v[pl.ds(0, 16)] = jnp.where(lane == 0, jnp.full((16,), acc), zf)
    pltpu.sync_copy(sum_v, part_hbm.at[wid])


def _run_topk_gather(sim, prompt_flat, yn):
    mesh = plsc.VectorSubcoreMesh(core_axis_name="c", subcore_axis_name="s",
                                  num_cores=_NC, num_subcores=_NS)
    fn = pl.kernel(
        _sc_body,
        out_type=[
            jax.ShapeDtypeStruct((_B, _HEAD, _D), jnp.float32),
            jax.ShapeDtypeStruct((_B, 128), jnp.int32),
            jax.ShapeDtypeStruct((_B, _K, _D), jnp.float32),
            jax.ShapeDtypeStruct((_NW, 128), jnp.float32),
        ],
        mesh=mesh,
        compiler_params=pltpu.CompilerParams(needs_layout_passes=False,
                                             use_tc_tiling_on_sc=True),
        scratch_types=[
            pltpu.VMEM((_ROWS_PER_W, _P), jnp.float32),
            pltpu.VMEM((_ROWS_PER_W, 128), jnp.int32),
            pltpu.VMEM((_ROWS_PER_W, 48), jnp.int32),
            pltpu.VMEM((2, _HEAD, _D), jnp.float32),
            pltpu.VMEM((_ROWS_PER_W, _K, _D), jnp.float32),
            pltpu.VMEM((128,), jnp.float32),
            pltpu.SemaphoreType.DMA,
            pltpu.SemaphoreType.DMA,
            pltpu.SemaphoreType.DMA,
            pltpu.SemaphoreType.DMA,
            pltpu.SemaphoreType.DMA,
            pltpu.SemaphoreType.DMA,
            pltpu.SemaphoreType.DMA,
        ],
    )
    return fn(sim, prompt_flat, yn)


_DBT = 32


def _assemble_body(big_in_ref, head_ref, part_ref, big_ref, rs_ref):
    del big_in_ref
    big_ref[...] = jnp.transpose(head_ref[...], (1, 0, 2))

    @pl.when(pl.program_id(0) == 0)
    def _():
        rs_ref[...] = jnp.sum(part_ref[...]).reshape(1, 1) / float(_B)


def _run_assemble(bigT0, head, part):
    return pl.pallas_call(
        _assemble_body,
        grid=(_B // _DBT,),
        in_specs=[
            pl.BlockSpec(memory_space=pl.ANY),
            pl.BlockSpec((_DBT, _HEAD, _D), lambda b: (b, 0, 0)),
            pl.BlockSpec((_NW, 128), lambda b: (0, 0)),
        ],
        out_specs=[
            pl.BlockSpec((_HEAD, _DBT, _D), lambda b: (0, b, 0)),
            pl.BlockSpec((1, 1), lambda b: (0, 0)),
        ],
        out_shape=[
            jax.ShapeDtypeStruct((_OUT_S, _B, _D), jnp.float32),
            jax.ShapeDtypeStruct((1, 1), jnp.float32),
        ],
        input_output_aliases={0: 0},
    )(bigT0, head, part)


def kernel(x_embed, prompt, prompt_key):
    xT = jnp.transpose(x_embed, (1, 0, 2))
    bigT0, sim, yn = _run_mean_copy_dist(xT, prompt_key)
    prompt_flat = jnp.transpose(prompt, (1, 0, 2)).reshape(_L * _P, _D)
    head, idx_pad, bkn, part = _run_topk_gather(sim, prompt_flat, yn)
    bigT, rs = _run_assemble(bigT0, head, part)
    big = jnp.transpose(bigT, (1, 0, 2))
    return big, rs[0, 0], sim, idx_pad[:, :_K], bkn


# speedup vs baseline: 1.2964x; 1.0172x over previous
import functools

import jax
import jax.numpy as jnp
from jax import lax
from jax.experimental import pallas as pl
from jax.experimental.pallas import tpu as pltpu
from jax.experimental.pallas import tpu_sc as plsc

_SCALE = 0.1
_K = 8
_L = 5
_P = 1024
_D = 768
_B = 128
_S = 196
_OUT_S = _K * _L + _S
_HEAD = _K * _L
_ROW_W = _L * _D

_NC = 2
_NS = 16
_NW = _NC * _NS
_ROWS_PER_W = _B // _NW
_CHUNKS = _P // 16


_ST = 4


def _map_to_ball(u):
    ss = jnp.sum(u * u, axis=-1, keepdims=True)
    un = u * lax.rsqrt(jnp.maximum(ss, 1e-12))
    us = un * _SCALE
    n2 = jnp.sum(us * us, axis=-1, keepdims=True)
    n = jnp.sqrt(jnp.maximum(n2, 1e-15))
    y = jnp.tanh(n) * us / n
    yn2 = jnp.sum(y * y, axis=-1, keepdims=True)
    ynorm = jnp.sqrt(jnp.maximum(yn2, 1e-15))
    maxnorm = 1.0 - 4e-3
    return jnp.where(ynorm > maxnorm, y / ynorm * maxnorm, y)


_SLABS = (4, 8, 16, 28, 42, 49, 49)
_AMAX = 49


def _mean_copy_dist_body(x_any, pk_ref, big_any, sim_ref, yn_ref,
                         xbuf, s_in0, s_in1, s_out0, s_out1):
    s_in = (s_in0, s_in1)
    s_out = (s_out0, s_out1)
    ns = len(_SLABS)
    offs = [sum(_SLABS[:k]) for k in range(ns)]
    h_in = [None] * ns
    h_out = [None] * ns

    h_in[0] = pltpu.make_async_copy(
        x_any.at[pl.ds(0, _SLABS[0])],
        xbuf.at[0, pl.ds(0, _SLABS[0])], s_in[0])
    h_in[0].start()
    total = jnp.zeros((_B, _D), jnp.float32)
    for k, sz in enumerate(_SLABS):
        slot = k % 2
        if k + 1 < ns:
            if k >= 1:
                h_out[k - 1].wait()
            h_in[k + 1] = pltpu.make_async_copy(
                x_any.at[pl.ds(offs[k + 1], _SLABS[k + 1])],
                xbuf.at[1 - slot, pl.ds(0, _SLABS[k + 1])], s_in[1 - slot])
            h_in[k + 1].start()
        h_in[k].wait()
        h_out[k] = pltpu.make_async_copy(
            xbuf.at[slot, pl.ds(0, sz)],
            big_any.at[pl.ds(_HEAD + offs[k], sz)], s_out[slot])
        h_out[k].start()

        def acc_row(i, s, slot=slot):
            return s + xbuf[slot, i]

        total = lax.fori_loop(0, sz, acc_row, total)

    if True:
        x = _map_to_ball(total / float(_S))
        y = _map_to_ball(pk_ref[...])
        yn_ref[...] = y
        x2 = jnp.sum(x * x, axis=-1, keepdims=True)
        y2 = jnp.sum(y * y, axis=-1)[None, :]
        xy = lax.dot_general(x, y, (((1,), (1,)), ((), ())),
                             precision=lax.Precision.HIGHEST,
                             preferred_element_type=jnp.float32)
        alpha = 1.0 - 2.0 * xy + y2
        beta = 1.0 - x2
        num2 = alpha * alpha * x2 + beta * beta * y2 - 2.0 * alpha * beta * xy
        den = 1.0 - 2.0 * xy + x2 * y2
        norm = jnp.sqrt(jnp.maximum(num2, 1e-15)) / (den + 1e-15)
        arg = jnp.clip(norm, 0.0, 1.0 - 1e-7)
        dist = jnp.log((1.0 + arg) / (1.0 - arg))
        sim_ref[...] = -dist

    h_out[ns - 2].wait()
    h_out[ns - 1].wait()


def _run_mean_copy_dist(xT, prompt_key):
    return pl.pallas_call(
        _mean_copy_dist_body,
        in_specs=[
            pl.BlockSpec(memory_space=pl.ANY),
            pl.BlockSpec((_P, _D)),
        ],
        out_specs=[
            pl.BlockSpec(memory_space=pl.ANY),
            pl.BlockSpec((_B, _P)),
            pl.BlockSpec((_P, _D)),
        ],
        out_shape=[
            jax.ShapeDtypeStruct((_OUT_S, _B, _D), jnp.float32),
            jax.ShapeDtypeStruct((_B, _P), jnp.float32),
            jax.ShapeDtypeStruct((_P, _D), jnp.float32),
        ],
        scratch_shapes=[
            pltpu.VMEM((2, _AMAX, _B, _D), jnp.float32),
            pltpu.SemaphoreType.DMA,
            pltpu.SemaphoreType.DMA,
            pltpu.SemaphoreType.DMA,
            pltpu.SemaphoreType.DMA,
        ],
    )(xT, prompt_key)


def _vgather16(v, i):
    dn = lax.GatherDimensionNumbers(offset_dims=(), collapsed_slice_dims=(0,),
                                    start_index_map=(0,))
    return lax.gather(v, i[:, None], dimension_numbers=dn, slice_sizes=(1,),
                      mode=lax.GatherScatterMode.PROMISE_IN_BOUNDS)


def _sc_body(sim_hbm, prompt_hbm, key_hbm,
             head_hbm, idx_hbm, bkn_hbm, part_hbm,
             sim_v, idx128s, idx40s, rows2, bkn4, sum_v,
             sP0, sP1, sH0, sH1, sK, sB, sI):
    cid = lax.axis_index("c")
    sid = lax.axis_index("s")
    wid = cid * _NS + sid
    lane = lax.iota(jnp.int32, 16)
    neg = jnp.full((16,), -3.0e38, jnp.float32)
    zidx = jnp.zeros((16,), jnp.int32)
    zf = jnp.zeros((16,), jnp.float32)
    base_b = wid * _ROWS_PER_W

    pltpu.sync_copy(sim_hbm.at[pl.ds(base_b, _ROWS_PER_W)], sim_v)

    sP = (sP0, sP1)
    sH = (sH0, sH1)
    nr = _ROWS_PER_W
    hP = [None] * nr
    hH = [None] * nr
    hK = [None] * nr
    hI = [None] * nr

    acc = jnp.float32(0.0)
    for r in range(nr):
        slot = r % 2

        def chunk_body(i, carry, r=r):
            bk, bi = carry
            cka = sim_v[r, pl.ds(i * 32, 16)]
            ckb = sim_v[r, pl.ds(i * 32 + 16, 16)]
            cia = lane + i * 32
            cib = cia + 16
            ska, sia = plsc.sort_key_val(cka, cia, descending=True)
            skb, sib = plsc.sort_key_val(ckb, cib, descending=True)
            pk_ = jnp.where(lane < 8, ska, lax.rev(skb, (0,)))
            pi_ = jnp.where(lane < 8, sia, lax.rev(sib, (0,)))
            spk, spi = plsc.sort_key_val(pk_, pi_, descending=True)
            mk = jnp.where(lane < 8, bk, lax.rev(spk, (0,)))
            mi = jnp.where(lane < 8, bi, lax.rev(spi, (0,)))
            return tuple(plsc.sort_key_val(mk, mi, descending=True))

        bk, bi = lax.fori_loop(0, _CHUNKS // 2, chunk_body, (neg, zidx))
        iv = jnp.where(lane < 8, bi, jnp.int32(2147483647))
        fi, fv = plsc.sort_key_val(iv, bk, descending=False)
        acc = acc - jnp.sum(jnp.where(lane < 8, fv, jnp.float32(0.0)))

        if r >= 2:
            hH[r - 2].wait()

        idx128s[r, pl.ds(0, 16)] = fi
        for c in range(3):
            j = lane + 16 * c
            q = lax.div(j, jnp.int32(_L))
            s = j - q * _L
            sel = _vgather16(fi, jnp.minimum(q, jnp.int32(15)))
            idx40s[r, pl.ds(16 * c, 16)] = s * _P + sel

        hI[r] = pltpu.async_copy(idx128s.at[r], idx_hbm.at[base_b + r], sI)
        hP[r] = pltpu.async_copy(prompt_hbm.at[idx40s.at[r, pl.ds(0, _HEAD)]],
                                 rows2.at[slot], sP[slot])
        hK[r] = pltpu.async_copy(key_hbm.at[idx128s.at[r, pl.ds(0, 8)]],
                                 bkn4.at[r], sK)
        if r >= 1:
            hP[r - 1].wait()
            hH[r - 1] = pltpu.async_copy(
                rows2.at[1 - slot], head_hbm.at[base_b + r - 1], sH[1 - slot])

    hP[nr - 1].wait()
    hH[nr - 1] = pltpu.async_copy(
        rows2.at[(nr - 1) % 2], head_hbm.at[base_b + nr - 1], sH[(nr - 1) % 2])
    for r in range(nr):
        hK[r].wait()
    hB = [None] * nr
    for r in range(nr):
        hB[r] = pltpu.async_copy(bkn4.at[r], bkn_hbm.at[base_b + r], sB)
    hH[nr - 2].wait()
    hH[nr - 1].wait()
    for r in range(nr):
        hB[r].wait()
        hI[r].wait()

    for c in range(8):
        sum_v[pl.ds(16 * c, 16)] = zf
    sum_v[pl.ds(0, 16)] = jnp.where(lane == 0, jnp.full((16,), acc), zf)
    pltpu.sync_copy(sum_v, part_hbm.at[wid])


def _run_topk_gather(sim, prompt_flat, yn):
    mesh = plsc.VectorSubcoreMesh(core_axis_name="c", subcore_axis_name="s",
                                  num_cores=_NC, num_subcores=_NS)
    fn = pl.kernel(
        _sc_body,
        out_type=[
            jax.ShapeDtypeStruct((_B, _HEAD, _D), jnp.float32),
            jax.ShapeDtypeStruct((_B, 128), jnp.int32),
            jax.ShapeDtypeStruct((_B, _K, _D), jnp.float32),
            jax.ShapeDtypeStruct((_NW, 128), jnp.float32),
        ],
        mesh=mesh,
        compiler_params=pltpu.CompilerParams(needs_layout_passes=False,
                                             use_tc_tiling_on_sc=True),
        scratch_types=[
            pltpu.VMEM((_ROWS_PER_W, _P), jnp.float32),
            pltpu.VMEM((_ROWS_PER_W, 128), jnp.int32),
            pltpu.VMEM((_ROWS_PER_W, 48), jnp.int32),
            pltpu.VMEM((2, _HEAD, _D), jnp.float32),
            pltpu.VMEM((_ROWS_PER_W, _K, _D), jnp.float32),
            pltpu.VMEM((128,), jnp.float32),
            pltpu.SemaphoreType.DMA,
            pltpu.SemaphoreType.DMA,
            pltpu.SemaphoreType.DMA,
            pltpu.SemaphoreType.DMA,
            pltpu.SemaphoreType.DMA,
            pltpu.SemaphoreType.DMA,
            pltpu.SemaphoreType.DMA,
        ],
    )
    return fn(sim, prompt_flat, yn)


_DBT = 32


def _assemble_body(big_in_ref, head_ref, part_ref, big_ref, rs_ref):
    del big_in_ref
    big_ref[...] = jnp.transpose(head_ref[...], (1, 0, 2))

    @pl.when(pl.program_id(0) == 0)
    def _():
        rs_ref[...] = jnp.sum(part_ref[...]).reshape(1, 1) / float(_B)


def _run_assemble(bigT0, head, part):
    return pl.pallas_call(
        _assemble_body,
        grid=(_B // _DBT,),
        in_specs=[
            pl.BlockSpec(memory_space=pl.ANY),
            pl.BlockSpec((_DBT, _HEAD, _D), lambda b: (b, 0, 0)),
            pl.BlockSpec((_NW, 128), lambda b: (0, 0)),
        ],
        out_specs=[
            pl.BlockSpec((_HEAD, _DBT, _D), lambda b: (0, b, 0)),
            pl.BlockSpec((1, 1), lambda b: (0, 0)),
        ],
        out_shape=[
            jax.ShapeDtypeStruct((_OUT_S, _B, _D), jnp.float32),
            jax.ShapeDtypeStruct((1, 1), jnp.float32),
        ],
        input_output_aliases={0: 0},
    )(bigT0, head, part)


def kernel(x_embed, prompt, prompt_key):
    xT = jnp.transpose(x_embed, (1, 0, 2))
    bigT0, sim, yn = _run_mean_copy_dist(xT, prompt_key)
    prompt_flat = jnp.transpose(prompt, (1, 0, 2)).reshape(_L * _P, _D)
    head, idx_pad, bkn, part = _run_topk_gather(sim, prompt_flat, yn)
    bigT, rs = _run_assemble(bigT0, head, part)
    big = jnp.transpose(bigT, (1, 0, 2))
    return big, rs[0, 0], sim, idx_pad[:, :_K], bkn


# speedup vs baseline: 1.2984x; 1.0015x over previous
import functools

import jax
import jax.numpy as jnp
from jax import lax
from jax.experimental import pallas as pl
from jax.experimental.pallas import tpu as pltpu
from jax.experimental.pallas import tpu_sc as plsc

_SCALE = 0.1
_K = 8
_L = 5
_P = 1024
_D = 768
_B = 128
_S = 196
_OUT_S = _K * _L + _S
_HEAD = _K * _L
_ROW_W = _L * _D

_NC = 2
_NS = 16
_NW = _NC * _NS
_ROWS_PER_W = _B // _NW
_CHUNKS = _P // 16


def _map_to_ball(u):
    ss = jnp.sum(u * u, axis=-1, keepdims=True)
    un = u * lax.rsqrt(jnp.maximum(ss, 1e-12))
    us = un * _SCALE
    n2 = jnp.sum(us * us, axis=-1, keepdims=True)
    n = jnp.sqrt(jnp.maximum(n2, 1e-15))
    y = jnp.tanh(n) * us / n
    yn2 = jnp.sum(y * y, axis=-1, keepdims=True)
    ynorm = jnp.sqrt(jnp.maximum(yn2, 1e-15))
    maxnorm = 1.0 - 4e-3
    return jnp.where(ynorm > maxnorm, y / ynorm * maxnorm, y)


_SLABS = (4, 8, 16, 28, 42, 49, 49)
_AMAX = 49


def _mean_copy_dist_body(x_any, pk_ref, big_any, sim_ref, yn_ref,
                         xbuf, s_in0, s_in1, s_out0, s_out1):
    s_in = (s_in0, s_in1)
    s_out = (s_out0, s_out1)
    ns = len(_SLABS)
    offs = [sum(_SLABS[:k]) for k in range(ns)]
    h_in = [None] * ns
    h_out = [None] * ns

    h_in[0] = pltpu.make_async_copy(
        x_any.at[pl.ds(0, _SLABS[0])],
        xbuf.at[0, pl.ds(0, _SLABS[0])], s_in[0])
    h_in[0].start()
    total = jnp.zeros((_B, _D), jnp.float32)
    for k, sz in enumerate(_SLABS):
        slot = k % 2
        if k + 1 < ns:
            if k >= 1:
                h_out[k - 1].wait()
            h_in[k + 1] = pltpu.make_async_copy(
                x_any.at[pl.ds(offs[k + 1], _SLABS[k + 1])],
                xbuf.at[1 - slot, pl.ds(0, _SLABS[k + 1])], s_in[1 - slot])
            h_in[k + 1].start()
        h_in[k].wait()
        h_out[k] = pltpu.make_async_copy(
            xbuf.at[slot, pl.ds(0, sz)],
            big_any.at[pl.ds(_HEAD + offs[k], sz)], s_out[slot])
        h_out[k].start()

        def acc_row(i, s, slot=slot):
            return s + xbuf[slot, i]

        total = lax.fori_loop(0, sz, acc_row, total)

    if True:
        x = _map_to_ball(total / float(_S))
        y = _map_to_ball(pk_ref[...])
        yn_ref[...] = y
        x2 = jnp.sum(x * x, axis=-1, keepdims=True)
        y2 = jnp.sum(y * y, axis=-1)[None, :]
        xy = lax.dot_general(x, y, (((1,), (1,)), ((), ())),
                             precision=lax.Precision.HIGHEST,
                             preferred_element_type=jnp.float32)
        alpha = 1.0 - 2.0 * xy + y2
        beta = 1.0 - x2
        num2 = alpha * alpha * x2 + beta * beta * y2 - 2.0 * alpha * beta * xy
        den = 1.0 - 2.0 * xy + x2 * y2
        norm = jnp.sqrt(jnp.maximum(num2, 1e-15)) / (den + 1e-15)
        arg = jnp.clip(norm, 0.0, 1.0 - 1e-7)
        dist = jnp.log((1.0 + arg) / (1.0 - arg))
        sim_ref[...] = -dist

    h_out[ns - 2].wait()
    h_out[ns - 1].wait()


def _run_mean_copy_dist(xT, prompt_key):
    return pl.pallas_call(
        _mean_copy_dist_body,
        in_specs=[
            pl.BlockSpec(memory_space=pl.ANY),
            pl.BlockSpec((_P, _D)),
        ],
        out_specs=[
            pl.BlockSpec(memory_space=pl.ANY),
            pl.BlockSpec((_B, _P)),
            pl.BlockSpec((_P, _D)),
        ],
        out_shape=[
            jax.ShapeDtypeStruct((_OUT_S, _B, _D), jnp.float32),
            jax.ShapeDtypeStruct((_B, _P), jnp.float32),
            jax.ShapeDtypeStruct((_P, _D), jnp.float32),
        ],
        scratch_shapes=[
            pltpu.VMEM((2, _AMAX, _B, _D), jnp.float32),
            pltpu.SemaphoreType.DMA,
            pltpu.SemaphoreType.DMA,
            pltpu.SemaphoreType.DMA,
            pltpu.SemaphoreType.DMA,
        ],
    )(xT, prompt_key)


def _vgather16(v, i):
    dn = lax.GatherDimensionNumbers(offset_dims=(), collapsed_slice_dims=(0,),
                                    start_index_map=(0,))
    return lax.gather(v, i[:, None], dimension_numbers=dn, slice_sizes=(1,),
                      mode=lax.GatherScatterMode.PROMISE_IN_BOUNDS)


def _sc_body(sim_hbm, prompt_hbm, key_hbm,
             head_hbm, idx_hbm, bkn_hbm, part_hbm,
             sim_v, idx128s, idx40s, rows2, bkn4, sum_v,
             sP0, sP1, sH0, sH1, sK, sB, sI):
    cid = lax.axis_index("c")
    sid = lax.axis_index("s")
    wid = cid * _NS + sid
    lane = lax.iota(jnp.int32, 16)
    neg = jnp.full((16,), -3.0e38, jnp.float32)
    zidx = jnp.zeros((16,), jnp.int32)
    zf = jnp.zeros((16,), jnp.float32)
    base_b = wid * _ROWS_PER_W

    pltpu.sync_copy(sim_hbm.at[pl.ds(base_b, _ROWS_PER_W)], sim_v)

    sP = (sP0, sP1)
    sH = (sH0, sH1)
    nr = _ROWS_PER_W
    hP = [None] * nr
    hH = [None] * nr
    hK = [None] * nr
    hI = [None] * nr

    acc = jnp.float32(0.0)
    for r in range(nr):
        slot = r % 2

        def chunk_body(i, carry, r=r):
            bk, bi = carry
            cka = sim_v[r, pl.ds(i * 32, 16)]
            ckb = sim_v[r, pl.ds(i * 32 + 16, 16)]
            cia = lane + i * 32
            cib = cia + 16
            ska, sia = plsc.sort_key_val(cka, cia, descending=True)
            skb, sib = plsc.sort_key_val(ckb, cib, descending=True)
            pk_ = jnp.where(lane < 8, ska, lax.rev(skb, (0,)))
            pi_ = jnp.where(lane < 8, sia, lax.rev(sib, (0,)))
            spk, spi = plsc.sort_key_val(pk_, pi_, descending=True)
            mk = jnp.where(lane < 8, bk, lax.rev(spk, (0,)))
            mi = jnp.where(lane < 8, bi, lax.rev(spi, (0,)))
            return tuple(plsc.sort_key_val(mk, mi, descending=True))

        bk, bi = lax.fori_loop(0, _CHUNKS // 2, chunk_body, (neg, zidx))
        iv = jnp.where(lane < 8, bi, jnp.int32(2147483647))
        fi, fv = plsc.sort_key_val(iv, bk, descending=False)
        acc = acc - jnp.sum(jnp.where(lane < 8, fv, jnp.float32(0.0)))

        if r >= 2:
            hH[r - 2].wait()

        idx128s[r, pl.ds(0, 16)] = fi
        for c in range(3):
            j = lane + 16 * c
            q = lax.div(j, jnp.int32(_L))
            s = j - q * _L
            sel = _vgather16(fi, jnp.minimum(q, jnp.int32(15)))
            idx40s[r, pl.ds(16 * c, 16)] = s * _P + sel

        hI[r] = pltpu.async_copy(idx128s.at[r], idx_hbm.at[base_b + r], sI)
        hP[r] = pltpu.async_copy(prompt_hbm.at[idx40s.at[r, pl.ds(0, _HEAD)]],
                                 rows2.at[slot], sP[slot])
        hK[r] = pltpu.async_copy(key_hbm.at[idx128s.at[r, pl.ds(0, 8)]],
                                 bkn4.at[r], sK)
        if r >= 1:
            hP[r - 1].wait()
            hH[r - 1] = pltpu.async_copy(
                rows2.at[1 - slot], head_hbm.at[base_b + r - 1], sH[1 - slot])

    hP[nr - 1].wait()
    hH[nr - 1] = pltpu.async_copy(
        rows2.at[(nr - 1) % 2], head_hbm.at[base_b + nr - 1], sH[(nr - 1) % 2])
    for r in range(nr):
        hK[r].wait()
    hB = [None] * nr
    for r in range(nr):
        hB[r] = pltpu.async_copy(bkn4.at[r], bkn_hbm.at[base_b + r], sB)
    hH[nr - 2].wait()
    hH[nr - 1].wait()
    for r in range(nr):
        hB[r].wait()
        hI[r].wait()

    for c in range(8):
        sum_v[pl.ds(16 * c, 16)] = zf
    sum_v[pl.ds(0, 16)] = jnp.where(lane == 0, jnp.full((16,), acc), zf)
    pltpu.sync_copy(sum_v, part_hbm.at[wid])


def _run_topk_gather(sim, prompt_flat, yn):
    mesh = plsc.VectorSubcoreMesh(core_axis_name="c", subcore_axis_name="s",
                                  num_cores=_NC, num_subcores=_NS)
    fn = pl.kernel(
        _sc_body,
        out_type=[
            jax.ShapeDtypeStruct((_B, _HEAD, _D), jnp.float32),
            jax.ShapeDtypeStruct((_B, 128), jnp.int32),
            jax.ShapeDtypeStruct((_B, _K, _D), jnp.float32),
            jax.ShapeDtypeStruct((_NW, 128), jnp.float32),
        ],
        mesh=mesh,
        compiler_params=pltpu.CompilerParams(needs_layout_passes=False,
                                             use_tc_tiling_on_sc=True),
        scratch_types=[
            pltpu.VMEM((_ROWS_PER_W, _P), jnp.float32),
            pltpu.VMEM((_ROWS_PER_W, 128), jnp.int32),
            pltpu.VMEM((_ROWS_PER_W, 48), jnp.int32),
            pltpu.VMEM((2, _HEAD, _D), jnp.float32),
            pltpu.VMEM((_ROWS_PER_W, _K, _D), jnp.float32),
            pltpu.VMEM((128,), jnp.float32),
            pltpu.SemaphoreType.DMA,
            pltpu.SemaphoreType.DMA,
            pltpu.SemaphoreType.DMA,
            pltpu.SemaphoreType.DMA,
            pltpu.SemaphoreType.DMA,
            pltpu.SemaphoreType.DMA,
            pltpu.SemaphoreType.DMA,
        ],
    )
    return fn(sim, prompt_flat, yn)


_DBT = 32


def _assemble_body(big_in_ref, head_ref, part_ref, big_ref, rs_ref):
    del big_in_ref
    big_ref[...] = jnp.transpose(head_ref[...], (1, 0, 2))

    @pl.when(pl.program_id(0) == 0)
    def _():
        rs_ref[...] = jnp.sum(part_ref[...]).reshape(1, 1) / float(_B)


def _run_assemble(bigT0, head, part):
    return pl.pallas_call(
        _assemble_body,
        grid=(_B // _DBT,),
        in_specs=[
            pl.BlockSpec(memory_space=pl.ANY),
            pl.BlockSpec((_DBT, _HEAD, _D), lambda b: (b, 0, 0)),
            pl.BlockSpec((_NW, 128), lambda b: (0, 0)),
        ],
        out_specs=[
            pl.BlockSpec((_HEAD, _DBT, _D), lambda b: (0, b, 0)),
            pl.BlockSpec((1, 1), lambda b: (0, 0)),
        ],
        out_shape=[
            jax.ShapeDtypeStruct((_OUT_S, _B, _D), jnp.float32),
            jax.ShapeDtypeStruct((1, 1), jnp.float32),
        ],
        input_output_aliases={0: 0},
    )(bigT0, head, part)


def kernel(x_embed, prompt, prompt_key):
    xT = jnp.transpose(x_embed, (1, 0, 2))
    bigT0, sim, yn = _run_mean_copy_dist(xT, prompt_key)
    prompt_flat = jnp.transpose(prompt, (1, 0, 2)).reshape(_L * _P, _D)
    head, idx_pad, bkn, part = _run_topk_gather(sim, prompt_flat, yn)
    bigT, rs = _run_assemble(bigT0, head, part)
    big = jnp.transpose(bigT, (1, 0, 2))
    return big, rs[0, 0], sim, idx_pad[:, :_K], bkn
